# baseline (speedup 1.0000x reference)
import functools

import jax
import jax.numpy as jnp
from jax import lax
from jax.experimental import pallas as pl
from jax.experimental.pallas import tpu as pltpu
from jax.experimental.pallas import tpu_sc as plsc

HID = 64
HP = 128
NTILES = 16
CHUNK = 128
RT = 528
SIZE_P = RT * NTILES


def _gelu(x):
    return jax.nn.gelu(x, approximate=True)


def _sc_mesh():
    return plsc.VectorSubcoreMesh(core_axis_name="c", subcore_axis_name="s")


_SC_PARAMS = pltpu.CompilerParams(needs_layout_passes=False,
                                  use_tc_tiling_on_sc=False)


def _index_kernel(vox, coords):
    Bn, NP = vox.shape
    NX = coords.shape[1]
    pts_per_tile = NP // NTILES
    nch = pts_per_tile // CHUNK
    steps = []
    st = NX
    while st >= 1:
        steps.append(st)
        st //= 2

    @functools.partial(
        pl.kernel,
        out_type=[
            jax.ShapeDtypeStruct((Bn, NP), jnp.int32),
            jax.ShapeDtypeStruct((Bn, NTILES, 1, RT), jnp.float32),
        ],
        mesh=_sc_mesh(),
        compiler_params=_SC_PARAMS,
        scratch_types=[
            pltpu.VMEM((NX,), jnp.int32),
            pltpu.VMEM((CHUNK,), jnp.int32),
            pltpu.VMEM((CHUNK,), jnp.int32),
            pltpu.VMEM((CHUNK, 16), jnp.float32),
            pltpu.VMEM((RT, 16), jnp.float32),
            pltpu.VMEM((1, RT), jnp.float32),
            pltpu.VMEM_SHARED((SIZE_P, 16), jnp.float32),
        ],
    )
    def k(vox_hbm, coords_hbm, index_hbm, invcnt_hbm,
          coords_v, vox_v, idx_v, ones_v, cnt_v, inv_v, cnt_sh):
        c = lax.axis_index("c")
        s = lax.axis_index("s")
        rslice = pl.ds(s * RT, RT)
        pltpu.sync_copy(coords_hbm.at[c], coords_v)

        def zero_body(r, carry):
            ones_v[r, :] = jnp.ones((16,), jnp.float32)
            cnt_v[r, :] = jnp.zeros((16,), jnp.float32)
            return carry

        lax.fori_loop(0, CHUNK, zero_body, 0)

        def zero_body2(r, carry):
            cnt_v[r, :] = jnp.zeros((16,), jnp.float32)
            return carry

        lax.fori_loop(CHUNK, RT, zero_body2, 0)
        pltpu.sync_copy(cnt_v, cnt_sh.at[rslice])
        plsc.subcore_barrier()
        base = s * pts_per_tile

        def chunk_body(ch, carry):
            off = pl.multiple_of(base + ch * CHUNK, CHUNK)
            pltpu.sync_copy(vox_hbm.at[c].at[pl.ds(off, CHUNK)], vox_v)
            for r in range(CHUNK // 16):
                v = vox_v[pl.ds(r * 16, 16)]
                pos = jnp.zeros((16,), jnp.int32)
                for st in steps:
                    nxt = pos + st
                    ok = nxt <= NX
                    probe = jnp.minimum(nxt - 1, NX - 1)
                    cv = plsc.load_gather(coords_v, [probe])
                    pos = jnp.where(ok & (cv < v), nxt, pos)
                idx_v[pl.ds(r * 16, 16)] = pos
            pltpu.sync_copy(idx_v, index_hbm.at[c].at[pl.ds(off, CHUNK)])
            pltpu.sync_copy(ones_v, cnt_sh.at[idx_v], add=True)
            return carry

        lax.fori_loop(0, nch, chunk_body, 0)
        plsc.subcore_barrier()
        pltpu.sync_copy(cnt_sh.at[rslice], cnt_v)

        def inv_body(g, carry):
            rows = g * 16 + lax.iota(jnp.int32, 16)
            cnt = plsc.load_gather(cnt_v, [rows, jnp.zeros((16,), jnp.int32)])
            inv_v[0, pl.ds(g * 16, 16)] = 1.0 / jnp.maximum(cnt, 1.0)
            return carry

        lax.fori_loop(0, RT // 16, inv_body, 0)
        pltpu.sync_copy(inv_v, invcnt_hbm.at[c].at[s])

    return k(vox, coords)


_STAGE = 256
_NSUB = _STAGE // CHUNK


def _pool_kernel(feat, index2d, invcnt):
    N = feat.shape[0]
    Bn = invcnt.shape[0]
    NP = N // Bn
    pts_per_tile = NP // NTILES
    nst = pts_per_tile // _STAGE
    H = HID

    @functools.partial(
        pl.kernel,
        out_type=jax.ShapeDtypeStruct((N, HP), jnp.float32),
        mesh=_sc_mesh(),
        compiler_params=_SC_PARAMS,
        scratch_types=[
            pltpu.VMEM((2, _NSUB, CHUNK), jnp.int32),
            pltpu.VMEM((2, _STAGE, H), jnp.float32),
            pltpu.VMEM((RT, H), jnp.float32),
            pltpu.VMEM((1, RT), jnp.float32),
            pltpu.VMEM_SHARED((SIZE_P, H), jnp.float32),
            pltpu.SemaphoreType.DMA,
            pltpu.SemaphoreType.DMA,
        ],
    )
    def k(feat_hbm, index_hbm, invcnt_hbm, z_hbm,
          idx_v, rows_v, acc_v, inv_v, tab_sh, sem0, sem1):
        c = lax.axis_index("c")
        s = lax.axis_index("s")
        sems = (sem0, sem1)
        rslice = pl.ds(s * RT, RT)

        def zero_body(r, carry):
            for q in range(H // 16):
                acc_v[r, pl.ds(q * 16, 16)] = jnp.zeros((16,), jnp.float32)
            return carry

        lax.fori_loop(0, RT, zero_body, 0)
        pltpu.sync_copy(acc_v, tab_sh.at[rslice])
        plsc.subcore_barrier()
        base = c * NP + s * pts_per_tile

        def _ld(st, b):
            off = pl.multiple_of(base + st * _STAGE, _STAGE)
            row = pl.multiple_of((base + st * _STAGE) // CHUNK, _NSUB)
            pltpu.async_copy(index_hbm.at[pl.ds(row, _NSUB)], idx_v.at[b],
                             sems[b])
            pltpu.async_copy(feat_hbm.at[pl.ds(off, _STAGE), pl.ds(0, H)],
                             rows_v.at[b], sems[b])

        def _ld_wait(st, b):
            off = pl.multiple_of(base + st * _STAGE, _STAGE)
            row = pl.multiple_of((base + st * _STAGE) // CHUNK, _NSUB)
            pltpu.make_async_copy(index_hbm.at[pl.ds(row, _NSUB)],
                                  idx_v.at[b], sems[b]).wait()
            pltpu.make_async_copy(feat_hbm.at[pl.ds(off, _STAGE),
                                              pl.ds(0, H)],
                                  rows_v.at[b], sems[b]).wait()

        _ld(0, 0)
        _ld(1, 1)

        def sc_body(g, carry):
            for b in range(2):
                st = g * 2 + b
                off = pl.multiple_of(base + st * _STAGE, _STAGE)
                _ld_wait(st, b)
                for j in range(_NSUB):
                    pltpu.sync_copy(
                        rows_v.at[b].at[pl.ds(j * CHUNK, CHUNK)],
                        tab_sh.at[idx_v.at[b].at[j]], add=True)
                pltpu.sync_copy(rows_v.at[b],
                                z_hbm.at[pl.ds(off, _STAGE), pl.ds(0, H)])
                nxt = st + 2

                @pl.when(nxt < nst)
                def _():
                    _ld(nxt, b)
            return carry

        lax.fori_loop(0, nst // 2, sc_body, 0)
        plsc.subcore_barrier()
        pltpu.sync_copy(tab_sh.at[rslice], acc_v)
        pltpu.sync_copy(invcnt_hbm.at[c].at[s], inv_v)

        def grp_body(g, carry):
            inv16 = inv_v[0, pl.ds(g * 16, 16)]
            for j in range(16):
                bc = jnp.full((16,), inv16[j], jnp.float32)
                r = g * 16 + j
                for q in range(H // 16):
                    cs = pl.ds(q * 16, 16)
                    acc_v[r, cs] = acc_v[r, cs] * bc
            return carry

        lax.fori_loop(0, RT // 16, grp_body, 0)
        pltpu.sync_copy(acc_v, tab_sh.at[rslice])
        plsc.subcore_barrier()

        def _gst(st, b):
            row = pl.multiple_of((base + st * _STAGE) // CHUNK, _NSUB)
            pltpu.sync_copy(index_hbm.at[pl.ds(row, _NSUB)], idx_v.at[b])
            for j in range(_NSUB):
                pltpu.async_copy(tab_sh.at[idx_v.at[b].at[j]],
                                 rows_v.at[b].at[pl.ds(j * CHUNK, CHUNK)],
                                 sems[b])

        def _gproc(st, b):
            for j in range(_NSUB):
                pltpu.make_async_copy(
                    tab_sh.at[idx_v.at[b].at[j]],
                    rows_v.at[b].at[pl.ds(j * CHUNK, CHUNK)],
                    sems[b]).wait()
            off = pl.multiple_of(base + st * _STAGE, _STAGE)
            pltpu.sync_copy(rows_v.at[b],
                            z_hbm.at[pl.ds(off, _STAGE), pl.ds(H, H)])

        _gst(0, 0)
        _gst(1, 1)

        def g_body(g, carry):
            for b in range(2):
                st = g * 2 + b
                _gproc(st, b)
                nxt = st + 2

                @pl.when(nxt < nst)
                def _():
                    _gst(nxt, b)
            return carry

        lax.fori_loop(0, nst // 2, g_body, 0)

    return k(feat, index2d, invcnt)


def _scatter_mean_kernel(feat, index2d, invcnt, NX):
    N = feat.shape[0]
    Bn = invcnt.shape[0]
    NP = N // Bn
    pts_per_tile = NP // NTILES
    nst = pts_per_tile // _STAGE
    H = HID
    tail = NX - (NTILES - 1) * RT
    assert 0 < tail <= RT

    @functools.partial(
        pl.kernel,
        out_type=jax.ShapeDtypeStruct((Bn * NX, H), jnp.float32),
        mesh=_sc_mesh(),
        compiler_params=_SC_PARAMS,
        scratch_types=[
            pltpu.VMEM((2, _NSUB, CHUNK), jnp.int32),
            pltpu.VMEM((2, _STAGE, H), jnp.float32),
            pltpu.VMEM((RT, H), jnp.float32),
            pltpu.VMEM((1, RT), jnp.float32),
            pltpu.VMEM_SHARED((SIZE_P, H), jnp.float32),
            pltpu.SemaphoreType.DMA,
            pltpu.SemaphoreType.DMA,
        ],
    )
    def k(feat_hbm, index_hbm, invcnt_hbm, mean_hbm,
          idx_v, rows_v, acc_v, inv_v, tab_sh, sem0, sem1):
        c = lax.axis_index("c")
        s = lax.axis_index("s")
        sems = (sem0, sem1)
        rslice = pl.ds(s * RT, RT)

        def zero_body(r, carry):
            for q in range(H // 16):
                acc_v[r, pl.ds(q * 16, 16)] = jnp.zeros((16,), jnp.float32)
            return carry

        lax.fori_loop(0, RT, zero_body, 0)
        pltpu.sync_copy(acc_v, tab_sh.at[rslice])
        plsc.subcore_barrier()
        base = c * NP + s * pts_per_tile

        def _ld(st, b):
            off = pl.multiple_of(base + st * _STAGE, _STAGE)
            row = pl.multiple_of((base + st * _STAGE) // CHUNK, _NSUB)
            pltpu.async_copy(index_hbm.at[pl.ds(row, _NSUB)], idx_v.at[b],
                             sems[b])
            pltpu.async_copy(feat_hbm.at[pl.ds(off, _STAGE), pl.ds(0, H)],
                             rows_v.at[b], sems[b])

        def _ld_wait(st, b):
            off = pl.multiple_of(base + st * _STAGE, _STAGE)
            row = pl.multiple_of((base + st * _STAGE) // CHUNK, _NSUB)
            pltpu.make_async_copy(index_hbm.at[pl.ds(row, _NSUB)],
                                  idx_v.at[b], sems[b]).wait()
            pltpu.make_async_copy(feat_hbm.at[pl.ds(off, _STAGE),
                                              pl.ds(0, H)],
                                  rows_v.at[b], sems[b]).wait()

        _ld(0, 0)
        _ld(1, 1)

        def sc_body(g, carry):
            for b in range(2):
                st = g * 2 + b
                _ld_wait(st, b)
                for j in range(_NSUB):
                    pltpu.sync_copy(
                        rows_v.at[b].at[pl.ds(j * CHUNK, CHUNK)],
                        tab_sh.at[idx_v.at[b].at[j]], add=True)
                nxt = st + 2

                @pl.when(nxt < nst)
                def _():
                    _ld(nxt, b)
            return carry

        lax.fori_loop(0, nst // 2, sc_body, 0)
        plsc.subcore_barrier()
        pltpu.sync_copy(tab_sh.at[rslice], acc_v)
        pltpu.sync_copy(invcnt_hbm.at[c].at[s], inv_v)

        def grp_body(g, carry):
            inv16 = inv_v[0, pl.ds(g * 16, 16)]
            for j in range(16):
                bc = jnp.full((16,), inv16[j], jnp.float32)
                r = g * 16 + j
                for q in range(H // 16):
                    cs = pl.ds(q * 16, 16)
                    acc_v[r, cs] = acc_v[r, cs] * bc
            return carry

        lax.fori_loop(0, RT // 16, grp_body, 0)

        @pl.when(s < NTILES - 1)
        def _():
            pltpu.sync_copy(acc_v, mean_hbm.at[pl.ds(c * NX + s * RT, RT)])

        @pl.when(s == NTILES - 1)
        def _():
            pltpu.sync_copy(acc_v.at[pl.ds(0, tail)],
                            mean_hbm.at[pl.ds(c * NX + s * RT, tail)])

    return k(feat, index2d, invcnt)


_TC_BLK = 16384


def _full_spec(shape):
    nd = len(shape)
    return pl.BlockSpec(shape, lambda i: (0,) * nd)


def _full_block_spec():
    return pl.BlockSpec((_TC_BLK, HP), lambda i: (i, 0))


def _tc_first(coordf, wp, bp, w0, b0, w1, b1, ws):
    N = coordf.shape[0]

    def body(cf_ref, wp_ref, bp_ref, w0_ref, b0_ref, w1_ref, b1_ref, ws_ref,
             out_ref):
        cf = cf_ref[...]
        pp = 2.0 * (cf - jnp.floor(cf) - 0.5)
        x = jnp.dot(pp, wp_ref[...],
                    preferred_element_type=jnp.float32) + bp_ref[...]
        h = jnp.dot(_gelu(x), w0_ref[...],
                    preferred_element_type=jnp.float32) + b0_ref[...]
        dx = jnp.dot(_gelu(h), w1_ref[...],
                     preferred_element_type=jnp.float32) + b1_ref[...]
        o = jnp.dot(x, ws_ref[...],
                    preferred_element_type=jnp.float32) + dx
        out_ref[...] = jnp.concatenate(
            [o, jnp.zeros((o.shape[0], HP - HID), jnp.float32)], axis=1)

    return pl.pallas_call(
        body,
        grid=(N // _TC_BLK,),
        in_specs=[
            pl.BlockSpec((_TC_BLK, 3), lambda i: (i, 0)),
            _full_spec(wp.shape), _full_spec(bp.shape),
            _full_spec(w0.shape), _full_spec(b0.shape),
            _full_spec(w1.shape), _full_spec(b1.shape),
            _full_spec(ws.shape),
        ],
        out_specs=_full_block_spec(),
        out_shape=jax.ShapeDtypeStruct((N, HP), jnp.float32),
    )(coordf, wp, bp, w0, b0, w1, b1, ws)


def _tc_block(z, w0, b0, w1, b1, ws, wc=None, bc=None):
    N = z.shape[0]
    final = wc is not None

    def body(*refs):
        z_ref, w0_ref, b0_ref, w1_ref, b1_ref, ws_ref = refs[:6]
        out_ref = refs[-1]
        x = z_ref[...]
        h = jnp.dot(_gelu(x), w0_ref[...],
                    preferred_element_type=jnp.float32) + b0_ref[...]
        dx = jnp.dot(_gelu(h), w1_ref[...],
                     preferred_element_type=jnp.float32) + b1_ref[...]
        o = jnp.dot(x, ws_ref[...],
                    preferred_element_type=jnp.float32) + dx
        if final:
            wc_ref, bc_ref = refs[6], refs[7]
            o = jnp.dot(o, wc_ref[...],
                        preferred_element_type=jnp.float32) + bc_ref[...]
        out_ref[...] = jnp.concatenate(
            [o, jnp.zeros((o.shape[0], HP - HID), jnp.float32)], axis=1)

    args = [z, w0, b0, w1, b1, ws]
    if final:
        args += [wc, bc]
    in_specs = [_full_block_spec()] + [_full_spec(a.shape) for a in args[1:]]
    return pl.pallas_call(
        body,
        grid=(N // _TC_BLK,),
        in_specs=in_specs,
        out_specs=_full_block_spec(),
        out_shape=jax.ShapeDtypeStruct((N, HP), jnp.float32),
    )(*args)


def kernel(p, sparse_coords, res, params):
    Bn, NP, _ = p.shape
    N = Bn * NP
    NX = sparse_coords.shape[0] // Bn

    dat = jnp.clip(p + 0.5, 1e-6, 1.0 - 1e-6)
    coord = dat * res
    ci = coord.astype(jnp.int32)
    vox = (ci[..., 0] * res + ci[..., 1]) * res + ci[..., 2]
    lin = (sparse_coords[:, 1] * res + sparse_coords[:, 2]) * res \
        + sparse_coords[:, 3]
    coords = lin.reshape(Bn, NX).astype(jnp.int32)
    coordf = coord.reshape(N, 3)

    index, invcnt = _index_kernel(vox, coords)
    index2d = index.reshape(N // CHUNK, CHUNK)

    Wp, bp = params["fc_pos"]
    bpr = bp.reshape(1, 2 * HID)

    W0, b0, W1, b1, Ws = params["blocks"][0]
    net = _tc_first(coordf, Wp.T, bpr, W0.T, b0.reshape(1, HID),
                    W1.T, b1.reshape(1, HID), Ws.T)

    Wc, bc = params["fc_c"]
    nblocks = len(params["blocks"])
    for i in range(1, nblocks):
        W0, b0, W1, b1, Ws = params["blocks"][i]
        z = _pool_kernel(net, index2d, invcnt)
        last = i == nblocks - 1
        net = _tc_block(z, W0.T, b0.reshape(1, HID),
                        W1.T, b1.reshape(1, HID), Ws.T,
                        wc=Wc.T if last else None,
                        bc=bc.reshape(1, HID) if last else None)

    return _scatter_mean_kernel(net, index2d, invcnt, NX)


# speedup vs baseline: 1.0044x; 1.0044x over previous
import functools

import jax
import jax.numpy as jnp
from jax import lax
from jax.experimental import pallas as pl
from jax.experimental.pallas import tpu as pltpu
from jax.experimental.pallas import tpu_sc as plsc

HID = 64
HP = 128
NTILES = 16
CHUNK = 128
RT = 528
SIZE_P = RT * NTILES


def _gelu(x):
    return jax.nn.gelu(x, approximate=True)


def _sc_mesh():
    return plsc.VectorSubcoreMesh(core_axis_name="c", subcore_axis_name="s")


_SC_PARAMS = pltpu.CompilerParams(needs_layout_passes=False,
                                  use_tc_tiling_on_sc=False)


def _index_kernel(vox, coords):
    Bn, NP = vox.shape
    NX = coords.shape[1]
    pts_per_tile = NP // NTILES
    nch = pts_per_tile // CHUNK
    steps = []
    st = NX
    while st >= 1:
        steps.append(st)
        st //= 2

    @functools.partial(
        pl.kernel,
        out_type=[
            jax.ShapeDtypeStruct((Bn, NP), jnp.int32),
            jax.ShapeDtypeStruct((Bn, NTILES, 1, RT), jnp.float32),
        ],
        mesh=_sc_mesh(),
        compiler_params=_SC_PARAMS,
        scratch_types=[
            pltpu.VMEM((NX,), jnp.int32),
            pltpu.VMEM((CHUNK,), jnp.int32),
            pltpu.VMEM((CHUNK,), jnp.int32),
            pltpu.VMEM((CHUNK, 16), jnp.float32),
            pltpu.VMEM((RT, 16), jnp.float32),
            pltpu.VMEM((1, RT), jnp.float32),
            pltpu.VMEM_SHARED((SIZE_P, 16), jnp.float32),
        ],
    )
    def k(vox_hbm, coords_hbm, index_hbm, invcnt_hbm,
          coords_v, vox_v, idx_v, ones_v, cnt_v, inv_v, cnt_sh):
        c = lax.axis_index("c")
        s = lax.axis_index("s")
        rslice = pl.ds(s * RT, RT)
        pltpu.sync_copy(coords_hbm.at[c], coords_v)

        def zero_body(r, carry):
            ones_v[r, :] = jnp.ones((16,), jnp.float32)
            cnt_v[r, :] = jnp.zeros((16,), jnp.float32)
            return carry

        lax.fori_loop(0, CHUNK, zero_body, 0)

        def zero_body2(r, carry):
            cnt_v[r, :] = jnp.zeros((16,), jnp.float32)
            return carry

        lax.fori_loop(CHUNK, RT, zero_body2, 0)
        pltpu.sync_copy(cnt_v, cnt_sh.at[rslice])
        plsc.subcore_barrier()
        base = s * pts_per_tile

        def chunk_body(ch, carry):
            off = pl.multiple_of(base + ch * CHUNK, CHUNK)
            pltpu.sync_copy(vox_hbm.at[c].at[pl.ds(off, CHUNK)], vox_v)
            for r in range(CHUNK // 16):
                v = vox_v[pl.ds(r * 16, 16)]
                pos = jnp.zeros((16,), jnp.int32)
                for st in steps:
                    nxt = pos + st
                    ok = nxt <= NX
                    probe = jnp.minimum(nxt - 1, NX - 1)
                    cv = plsc.load_gather(coords_v, [probe])
                    pos = jnp.where(ok & (cv < v), nxt, pos)
                idx_v[pl.ds(r * 16, 16)] = pos
            pltpu.sync_copy(idx_v, index_hbm.at[c].at[pl.ds(off, CHUNK)])
            pltpu.sync_copy(ones_v, cnt_sh.at[idx_v], add=True)
            return carry

        lax.fori_loop(0, nch, chunk_body, 0)
        plsc.subcore_barrier()
        pltpu.sync_copy(cnt_sh.at[rslice], cnt_v)

        def inv_body(g, carry):
            rows = g * 16 + lax.iota(jnp.int32, 16)
            cnt = plsc.load_gather(cnt_v, [rows, jnp.zeros((16,), jnp.int32)])
            inv_v[0, pl.ds(g * 16, 16)] = 1.0 / jnp.maximum(cnt, 1.0)
            return carry

        lax.fori_loop(0, RT // 16, inv_body, 0)
        pltpu.sync_copy(inv_v, invcnt_hbm.at[c].at[s])

    return k(vox, coords)


_STAGE = 256
_NSUB = _STAGE // CHUNK


def _pool_kernel(feat, index2d, invcnt):
    N = feat.shape[0]
    Bn = invcnt.shape[0]
    NP = N // Bn
    pts_per_tile = NP // NTILES
    nst = pts_per_tile // _STAGE
    H = HID

    @functools.partial(
        pl.kernel,
        out_type=jax.ShapeDtypeStruct((N, HP), jnp.float32),
        mesh=_sc_mesh(),
        compiler_params=_SC_PARAMS,
        scratch_types=[
            pltpu.VMEM((2, _NSUB, CHUNK), jnp.int32),
            pltpu.VMEM((2, _STAGE, H), jnp.float32),
            pltpu.VMEM((RT, H), jnp.float32),
            pltpu.VMEM((1, RT), jnp.float32),
            pltpu.VMEM_SHARED((SIZE_P, H), jnp.float32),
            pltpu.SemaphoreType.DMA,
            pltpu.SemaphoreType.DMA,
        ],
    )
    def k(feat_hbm, index_hbm, invcnt_hbm, z_hbm,
          idx_v, rows_v, acc_v, inv_v, tab_sh, sem0, sem1):
        c = lax.axis_index("c")
        s = lax.axis_index("s")
        sems = (sem0, sem1)
        rslice = pl.ds(s * RT, RT)

        def zero_body(r, carry):
            for q in range(H // 16):
                acc_v[r, pl.ds(q * 16, 16)] = jnp.zeros((16,), jnp.float32)
            return carry

        lax.fori_loop(0, RT, zero_body, 0)
        pltpu.sync_copy(acc_v, tab_sh.at[rslice])
        plsc.subcore_barrier()
        base = c * NP + s * pts_per_tile

        def _ld(st, b):
            off = pl.multiple_of(base + st * _STAGE, _STAGE)
            row = pl.multiple_of((base + st * _STAGE) // CHUNK, _NSUB)
            pltpu.async_copy(index_hbm.at[pl.ds(row, _NSUB)], idx_v.at[b],
                             sems[b])
            pltpu.async_copy(feat_hbm.at[pl.ds(off, _STAGE), pl.ds(0, H)],
                             rows_v.at[b], sems[b])

        def _ld_wait(st, b):
            off = pl.multiple_of(base + st * _STAGE, _STAGE)
            row = pl.multiple_of((base + st * _STAGE) // CHUNK, _NSUB)
            pltpu.make_async_copy(index_hbm.at[pl.ds(row, _NSUB)],
                                  idx_v.at[b], sems[b]).wait()
            pltpu.make_async_copy(feat_hbm.at[pl.ds(off, _STAGE),
                                              pl.ds(0, H)],
                                  rows_v.at[b], sems[b]).wait()

        _ld(0, 0)
        _ld(1, 1)

        def sc_body(g, carry):
            for b in range(2):
                st = g * 2 + b
                off = pl.multiple_of(base + st * _STAGE, _STAGE)
                _ld_wait(st, b)
                for j in range(_NSUB):
                    pltpu.sync_copy(
                        rows_v.at[b].at[pl.ds(j * CHUNK, CHUNK)],
                        tab_sh.at[idx_v.at[b].at[j]], add=True)
                pltpu.sync_copy(rows_v.at[b],
                                z_hbm.at[pl.ds(off, _STAGE), pl.ds(0, H)])
                nxt = st + 2

                @pl.when(nxt < nst)
                def _():
                    _ld(nxt, b)
            return carry

        lax.fori_loop(0, nst // 2, sc_body, 0)
        plsc.subcore_barrier()
        pltpu.sync_copy(tab_sh.at[rslice], acc_v)
        pltpu.sync_copy(invcnt_hbm.at[c].at[s], inv_v)

        def grp_body(g, carry):
            inv16 = inv_v[0, pl.ds(g * 16, 16)]
            for j in range(16):
                bc = jnp.full((16,), inv16[j], jnp.float32)
                r = g * 16 + j
                for q in range(H // 16):
                    cs = pl.ds(q * 16, 16)
                    acc_v[r, cs] = acc_v[r, cs] * bc
            return carry

        lax.fori_loop(0, RT // 16, grp_body, 0)
        pltpu.sync_copy(acc_v, tab_sh.at[rslice])
        plsc.subcore_barrier()

        def _gst(st, b):
            row = pl.multiple_of((base + st * _STAGE) // CHUNK, _NSUB)
            pltpu.sync_copy(index_hbm.at[pl.ds(row, _NSUB)], idx_v.at[b])
            for j in range(_NSUB):
                pltpu.async_copy(tab_sh.at[idx_v.at[b].at[j]],
                                 rows_v.at[b].at[pl.ds(j * CHUNK, CHUNK)],
                                 sems[b])

        def _gproc(st, b):
            for j in range(_NSUB):
                pltpu.make_async_copy(
                    tab_sh.at[idx_v.at[b].at[j]],
                    rows_v.at[b].at[pl.ds(j * CHUNK, CHUNK)],
                    sems[b]).wait()
            off = pl.multiple_of(base + st * _STAGE, _STAGE)
            pltpu.sync_copy(rows_v.at[b],
                            z_hbm.at[pl.ds(off, _STAGE), pl.ds(H, H)])

        _gst(0, 0)
        _gst(1, 1)

        def g_body(g, carry):
            for b in range(2):
                st = g * 2 + b
                _gproc(st, b)
                nxt = st + 2

                @pl.when(nxt < nst)
                def _():
                    _gst(nxt, b)
            return carry

        lax.fori_loop(0, nst // 2, g_body, 0)

    return k(feat, index2d, invcnt)


def _scatter_mean_kernel(feat, index2d, invcnt, NX):
    N = feat.shape[0]
    Bn = invcnt.shape[0]
    NP = N // Bn
    pts_per_tile = NP // NTILES
    nst = pts_per_tile // _STAGE
    H = HID
    tail = NX - (NTILES - 1) * RT
    assert 0 < tail <= RT

    @functools.partial(
        pl.kernel,
        out_type=jax.ShapeDtypeStruct((Bn * NX, H), jnp.float32),
        mesh=_sc_mesh(),
        compiler_params=_SC_PARAMS,
        scratch_types=[
            pltpu.VMEM((2, _NSUB, CHUNK), jnp.int32),
            pltpu.VMEM((2, _STAGE, H), jnp.float32),
            pltpu.VMEM((RT, H), jnp.float32),
            pltpu.VMEM((1, RT), jnp.float32),
            pltpu.VMEM_SHARED((SIZE_P, H), jnp.float32),
            pltpu.SemaphoreType.DMA,
            pltpu.SemaphoreType.DMA,
        ],
    )
    def k(feat_hbm, index_hbm, invcnt_hbm, mean_hbm,
          idx_v, rows_v, acc_v, inv_v, tab_sh, sem0, sem1):
        c = lax.axis_index("c")
        s = lax.axis_index("s")
        sems = (sem0, sem1)
        rslice = pl.ds(s * RT, RT)

        def zero_body(r, carry):
            for q in range(H // 16):
                acc_v[r, pl.ds(q * 16, 16)] = jnp.zeros((16,), jnp.float32)
            return carry

        lax.fori_loop(0, RT, zero_body, 0)
        pltpu.sync_copy(acc_v, tab_sh.at[rslice])
        plsc.subcore_barrier()
        base = c * NP + s * pts_per_tile

        def _ld(st, b):
            off = pl.multiple_of(base + st * _STAGE, _STAGE)
            row = pl.multiple_of((base + st * _STAGE) // CHUNK, _NSUB)
            pltpu.async_copy(index_hbm.at[pl.ds(row, _NSUB)], idx_v.at[b],
                             sems[b])
            pltpu.async_copy(feat_hbm.at[pl.ds(off, _STAGE), pl.ds(0, H)],
                             rows_v.at[b], sems[b])

        def _ld_wait(st, b):
            off = pl.multiple_of(base + st * _STAGE, _STAGE)
            row = pl.multiple_of((base + st * _STAGE) // CHUNK, _NSUB)
            pltpu.make_async_copy(index_hbm.at[pl.ds(row, _NSUB)],
                                  idx_v.at[b], sems[b]).wait()
            pltpu.make_async_copy(feat_hbm.at[pl.ds(off, _STAGE),
                                              pl.ds(0, H)],
                                  rows_v.at[b], sems[b]).wait()

        _ld(0, 0)
        _ld(1, 1)

        def sc_body(g, carry):
            for b in range(2):
                st = g * 2 + b
                _ld_wait(st, b)
                for j in range(_NSUB):
                    pltpu.sync_copy(
                        rows_v.at[b].at[pl.ds(j * CHUNK, CHUNK)],
                        tab_sh.at[idx_v.at[b].at[j]], add=True)
                nxt = st + 2

                @pl.when(nxt < nst)
                def _():
                    _ld(nxt, b)
            return carry

        lax.fori_loop(0, nst // 2, sc_body, 0)
        plsc.subcore_barrier()
        pltpu.sync_copy(tab_sh.at[rslice], acc_v)
        pltpu.sync_copy(invcnt_hbm.at[c].at[s], inv_v)

        def grp_body(g, carry):
            inv16 = inv_v[0, pl.ds(g * 16, 16)]
            for j in range(16):
                bc = jnp.full((16,), inv16[j], jnp.float32)
                r = g * 16 + j
                for q in range(H // 16):
                    cs = pl.ds(q * 16, 16)
                    acc_v[r, cs] = acc_v[r, cs] * bc
            return carry

        lax.fori_loop(0, RT // 16, grp_body, 0)

        @pl.when(s < NTILES - 1)
        def _():
            pltpu.sync_copy(acc_v, mean_hbm.at[pl.ds(c * NX + s * RT, RT)])

        @pl.when(s == NTILES - 1)
        def _():
            pltpu.sync_copy(acc_v.at[pl.ds(0, tail)],
                            mean_hbm.at[pl.ds(c * NX + s * RT, tail)])

    return k(feat, index2d, invcnt)


_TC_BLK = 8192


def _full_spec(shape):
    nd = len(shape)
    return pl.BlockSpec(shape, lambda i: (0,) * nd)


def _full_block_spec():
    return pl.BlockSpec((_TC_BLK, HP), lambda i: (i, 0))


def _tc_first(coordf, wp, bp, w0, b0, w1, b1, ws):
    N = coordf.shape[0]

    def body(cf_ref, wp_ref, bp_ref, w0_ref, b0_ref, w1_ref, b1_ref, ws_ref,
             out_ref):
        cf = cf_ref[...]
        pp = 2.0 * (cf - jnp.floor(cf) - 0.5)
        x = jnp.dot(pp, wp_ref[...],
                    preferred_element_type=jnp.float32) + bp_ref[...]
        h = jnp.dot(_gelu(x), w0_ref[...],
                    preferred_element_type=jnp.float32) + b0_ref[...]
        dx = jnp.dot(_gelu(h), w1_ref[...],
                     preferred_element_type=jnp.float32) + b1_ref[...]
        o = jnp.dot(x, ws_ref[...],
                    preferred_element_type=jnp.float32) + dx
        out_ref[...] = jnp.concatenate(
            [o, jnp.zeros((o.shape[0], HP - HID), jnp.float32)], axis=1)

    return pl.pallas_call(
        body,
        grid=(N // _TC_BLK,),
        in_specs=[
            pl.BlockSpec((_TC_BLK, 3), lambda i: (i, 0)),
            _full_spec(wp.shape), _full_spec(bp.shape),
            _full_spec(w0.shape), _full_spec(b0.shape),
            _full_spec(w1.shape), _full_spec(b1.shape),
            _full_spec(ws.shape),
        ],
        out_specs=_full_block_spec(),
        out_shape=jax.ShapeDtypeStruct((N, HP), jnp.float32),
    )(coordf, wp, bp, w0, b0, w1, b1, ws)


def _tc_block(z, w0, b0, w1, b1, ws, wc=None, bc=None):
    N = z.shape[0]
    final = wc is not None

    def body(*refs):
        z_ref, w0_ref, b0_ref, w1_ref, b1_ref, ws_ref = refs[:6]
        out_ref = refs[-1]
        x = z_ref[...]
        h = jnp.dot(_gelu(x), w0_ref[...],
                    preferred_element_type=jnp.float32) + b0_ref[...]
        dx = jnp.dot(_gelu(h), w1_ref[...],
                     preferred_element_type=jnp.float32) + b1_ref[...]
        o = jnp.dot(x, ws_ref[...],
                    preferred_element_type=jnp.float32) + dx
        if final:
            wc_ref, bc_ref = refs[6], refs[7]
            o = jnp.dot(o, wc_ref[...],
                        preferred_element_type=jnp.float32) + bc_ref[...]
        out_ref[...] = jnp.concatenate(
            [o, jnp.zeros((o.shape[0], HP - HID), jnp.float32)], axis=1)

    args = [z, w0, b0, w1, b1, ws]
    if final:
        args += [wc, bc]
    in_specs = [_full_block_spec()] + [_full_spec(a.shape) for a in args[1:]]
    return pl.pallas_call(
        body,
        grid=(N // _TC_BLK,),
        in_specs=in_specs,
        out_specs=_full_block_spec(),
        out_shape=jax.ShapeDtypeStruct((N, HP), jnp.float32),
    )(*args)


def kernel(p, sparse_coords, res, params):
    Bn, NP, _ = p.shape
    N = Bn * NP
    NX = sparse_coords.shape[0] // Bn

    dat = jnp.clip(p + 0.5, 1e-6, 1.0 - 1e-6)
    coord = dat * res
    ci = coord.astype(jnp.int32)
    vox = (ci[..., 0] * res + ci[..., 1]) * res + ci[..., 2]
    lin = (sparse_coords[:, 1] * res + sparse_coords[:, 2]) * res \
        + sparse_coords[:, 3]
    coords = lin.reshape(Bn, NX).astype(jnp.int32)
    coordf = coord.reshape(N, 3)

    index, invcnt = _index_kernel(vox, coords)
    index2d = index.reshape(N // CHUNK, CHUNK)

    Wp, bp = params["fc_pos"]
    bpr = bp.reshape(1, 2 * HID)

    W0, b0, W1, b1, Ws = params["blocks"][0]
    net = _tc_first(coordf, Wp.T, bpr, W0.T, b0.reshape(1, HID),
                    W1.T, b1.reshape(1, HID), Ws.T)

    Wc, bc = params["fc_c"]
    nblocks = len(params["blocks"])
    for i in range(1, nblocks):
        W0, b0, W1, b1, Ws = params["blocks"][i]
        z = _pool_kernel(net, index2d, invcnt)
        last = i == nblocks - 1
        net = _tc_block(z, W0.T, b0.reshape(1, HID),
                        W1.T, b1.reshape(1, HID), Ws.T,
                        wc=Wc.T if last else None,
                        bc=bc.reshape(1, HID) if last else None)

    return _scatter_mean_kernel(net, index2d, invcnt, NX)


# speedup vs baseline: 1.0482x; 1.0436x over previous
import functools

import jax
import jax.numpy as jnp
from jax import lax
from jax.experimental import pallas as pl
from jax.experimental.pallas import tpu as pltpu
from jax.experimental.pallas import tpu_sc as plsc

HID = 64
HP = 128
NTILES = 16
CHUNK = 128
RT = 528
SIZE_P = RT * NTILES


def _gelu(x):
    return jax.nn.gelu(x, approximate=True)


def _sc_mesh():
    return plsc.VectorSubcoreMesh(core_axis_name="c", subcore_axis_name="s")


_SC_PARAMS = pltpu.CompilerParams(needs_layout_passes=False,
                                  use_tc_tiling_on_sc=False)


def _index_kernel(vox, coords):
    Bn, NP = vox.shape
    NX = coords.shape[1]
    pts_per_tile = NP // NTILES
    nch = pts_per_tile // CHUNK
    steps = []
    st = NX
    while st >= 1:
        steps.append(st)
        st //= 2

    @functools.partial(
        pl.kernel,
        out_type=[
            jax.ShapeDtypeStruct((Bn, NP), jnp.int32),
            jax.ShapeDtypeStruct((Bn, NTILES, 1, RT), jnp.float32),
        ],
        mesh=_sc_mesh(),
        compiler_params=_SC_PARAMS,
        scratch_types=[
            pltpu.VMEM((NX,), jnp.int32),
            pltpu.VMEM((CHUNK,), jnp.int32),
            pltpu.VMEM((CHUNK,), jnp.int32),
            pltpu.VMEM((CHUNK, 16), jnp.float32),
            pltpu.VMEM((RT, 16), jnp.float32),
            pltpu.VMEM((1, RT), jnp.float32),
            pltpu.VMEM_SHARED((SIZE_P, 16), jnp.float32),
        ],
    )
    def k(vox_hbm, coords_hbm, index_hbm, invcnt_hbm,
          coords_v, vox_v, idx_v, ones_v, cnt_v, inv_v, cnt_sh):
        c = lax.axis_index("c")
        s = lax.axis_index("s")
        rslice = pl.ds(s * RT, RT)
        pltpu.sync_copy(coords_hbm.at[c], coords_v)

        def zero_body(r, carry):
            ones_v[r, :] = jnp.ones((16,), jnp.float32)
            cnt_v[r, :] = jnp.zeros((16,), jnp.float32)
            return carry

        lax.fori_loop(0, CHUNK, zero_body, 0)

        def zero_body2(r, carry):
            cnt_v[r, :] = jnp.zeros((16,), jnp.float32)
            return carry

        lax.fori_loop(CHUNK, RT, zero_body2, 0)
        pltpu.sync_copy(cnt_v, cnt_sh.at[rslice])
        plsc.subcore_barrier()
        base = s * pts_per_tile

        def chunk_body(ch, carry):
            off = pl.multiple_of(base + ch * CHUNK, CHUNK)
            pltpu.sync_copy(vox_hbm.at[c].at[pl.ds(off, CHUNK)], vox_v)
            for r in range(CHUNK // 16):
                v = vox_v[pl.ds(r * 16, 16)]
                pos = jnp.zeros((16,), jnp.int32)
                for st in steps:
                    nxt = pos + st
                    ok = nxt <= NX
                    probe = jnp.minimum(nxt - 1, NX - 1)
                    cv = plsc.load_gather(coords_v, [probe])
                    pos = jnp.where(ok & (cv < v), nxt, pos)
                idx_v[pl.ds(r * 16, 16)] = pos
            pltpu.sync_copy(idx_v, index_hbm.at[c].at[pl.ds(off, CHUNK)])
            pltpu.sync_copy(ones_v, cnt_sh.at[idx_v], add=True)
            return carry

        lax.fori_loop(0, nch, chunk_body, 0)
        plsc.subcore_barrier()
        pltpu.sync_copy(cnt_sh.at[rslice], cnt_v)

        def inv_body(g, carry):
            rows = g * 16 + lax.iota(jnp.int32, 16)
            cnt = plsc.load_gather(cnt_v, [rows, jnp.zeros((16,), jnp.int32)])
            inv_v[0, pl.ds(g * 16, 16)] = 1.0 / jnp.maximum(cnt, 1.0)
            return carry

        lax.fori_loop(0, RT // 16, inv_body, 0)
        pltpu.sync_copy(inv_v, invcnt_hbm.at[c].at[s])

    return k(vox, coords)


_STAGE = 256
_NSUB = _STAGE // CHUNK


def _pool_kernel(feat, index2d, invcnt):
    N = feat.shape[0]
    Bn = invcnt.shape[0]
    NP = N // Bn
    pts_per_tile = NP // NTILES
    nst = pts_per_tile // _STAGE
    H = HID

    @functools.partial(
        pl.kernel,
        out_type=jax.ShapeDtypeStruct((N, HP), jnp.float32),
        mesh=_sc_mesh(),
        compiler_params=_SC_PARAMS,
        scratch_types=[
            pltpu.VMEM((2, _NSUB, CHUNK), jnp.int32),
            pltpu.VMEM((2, _STAGE, H), jnp.float32),
            pltpu.VMEM((RT, H), jnp.float32),
            pltpu.VMEM((1, RT), jnp.float32),
            pltpu.VMEM_SHARED((SIZE_P, H), jnp.float32),
            pltpu.SemaphoreType.DMA,
            pltpu.SemaphoreType.DMA,
        ],
    )
    def k(feat_hbm, index_hbm, invcnt_hbm, z_hbm,
          idx_v, rows_v, acc_v, inv_v, tab_sh, sem0, sem1):
        c = lax.axis_index("c")
        s = lax.axis_index("s")
        sems = (sem0, sem1)
        rslice = pl.ds(s * RT, RT)

        def zero_body(r, carry):
            for q in range(H // 16):
                acc_v[r, pl.ds(q * 16, 16)] = jnp.zeros((16,), jnp.float32)
            return carry

        lax.fori_loop(0, RT, zero_body, 0)
        pltpu.sync_copy(acc_v, tab_sh.at[rslice])
        plsc.subcore_barrier()
        base = c * NP + s * pts_per_tile

        def _ld(st, b):
            off = pl.multiple_of(base + st * _STAGE, _STAGE)
            row = pl.multiple_of((base + st * _STAGE) // CHUNK, _NSUB)
            pltpu.async_copy(index_hbm.at[pl.ds(row, _NSUB)], idx_v.at[b],
                             sems[b])
            pltpu.async_copy(feat_hbm.at[pl.ds(off, _STAGE), pl.ds(0, H)],
                             rows_v.at[b], sems[b])

        def _ld_wait(st, b):
            off = pl.multiple_of(base + st * _STAGE, _STAGE)
            row = pl.multiple_of((base + st * _STAGE) // CHUNK, _NSUB)
            pltpu.make_async_copy(index_hbm.at[pl.ds(row, _NSUB)],
                                  idx_v.at[b], sems[b]).wait()
            pltpu.make_async_copy(feat_hbm.at[pl.ds(off, _STAGE),
                                              pl.ds(0, H)],
                                  rows_v.at[b], sems[b]).wait()

        _ld(0, 0)
        _ld(1, 1)

        def sc_body(g, carry):
            for b in range(2):
                st = g * 2 + b
                off = pl.multiple_of(base + st * _STAGE, _STAGE)
                _ld_wait(st, b)
                for j in range(_NSUB):
                    pltpu.sync_copy(
                        rows_v.at[b].at[pl.ds(j * CHUNK, CHUNK)],
                        tab_sh.at[idx_v.at[b].at[j]], add=True)
                pltpu.sync_copy(rows_v.at[b],
                                z_hbm.at[pl.ds(off, _STAGE), pl.ds(0, H)])
                nxt = st + 2

                @pl.when(nxt < nst)
                def _():
                    _ld(nxt, b)
            return carry

        lax.fori_loop(0, nst // 2, sc_body, 0)
        plsc.subcore_barrier()
        pltpu.sync_copy(tab_sh.at[rslice], acc_v)
        pltpu.sync_copy(invcnt_hbm.at[c].at[s], inv_v)

        def grp_body(g, carry):
            inv16 = inv_v[0, pl.ds(g * 16, 16)]
            for j in range(16):
                bc = jnp.full((16,), inv16[j], jnp.float32)
                r = g * 16 + j
                for q in range(H // 16):
                    cs = pl.ds(q * 16, 16)
                    acc_v[r, cs] = acc_v[r, cs] * bc
            return carry

        lax.fori_loop(0, RT // 16, grp_body, 0)
        pltpu.sync_copy(acc_v, tab_sh.at[rslice])
        plsc.subcore_barrier()

        def _gst(st, b):
            row = pl.multiple_of((base + st * _STAGE) // CHUNK, _NSUB)
            pltpu.sync_copy(index_hbm.at[pl.ds(row, _NSUB)], idx_v.at[b])
            for j in range(_NSUB):
                pltpu.async_copy(tab_sh.at[idx_v.at[b].at[j]],
                                 rows_v.at[b].at[pl.ds(j * CHUNK, CHUNK)],
                                 sems[b])

        def _gproc(st, b):
            for j in range(_NSUB):
                pltpu.make_async_copy(
                    tab_sh.at[idx_v.at[b].at[j]],
                    rows_v.at[b].at[pl.ds(j * CHUNK, CHUNK)],
                    sems[b]).wait()
            off = pl.multiple_of(base + st * _STAGE, _STAGE)
            pltpu.sync_copy(rows_v.at[b],
                            z_hbm.at[pl.ds(off, _STAGE), pl.ds(H, H)])

        _gst(0, 0)
        _gst(1, 1)

        def g_body(g, carry):
            for b in range(2):
                st = g * 2 + b
                _gproc(st, b)
                nxt = st + 2

                @pl.when(nxt < nst)
                def _():
                    _gst(nxt, b)
            return carry

        lax.fori_loop(0, nst // 2, g_body, 0)

    return k(feat, index2d, invcnt)


def _scatter_mean_kernel(feat, index2d, invcnt, NX):
    N = feat.shape[0]
    Bn = invcnt.shape[0]
    NP = N // Bn
    pts_per_tile = NP // NTILES
    nst = pts_per_tile // _STAGE
    H = HID
    tail = NX - (NTILES - 1) * RT
    assert 0 < tail <= RT

    @functools.partial(
        pl.kernel,
        out_type=jax.ShapeDtypeStruct((Bn * NX, H), jnp.float32),
        mesh=_sc_mesh(),
        compiler_params=_SC_PARAMS,
        scratch_types=[
            pltpu.VMEM((2, _NSUB, CHUNK), jnp.int32),
            pltpu.VMEM((2, _STAGE, H), jnp.float32),
            pltpu.VMEM((RT, H), jnp.float32),
            pltpu.VMEM((1, RT), jnp.float32),
            pltpu.VMEM_SHARED((SIZE_P, H), jnp.float32),
            pltpu.SemaphoreType.DMA,
            pltpu.SemaphoreType.DMA,
        ],
    )
    def k(feat_hbm, index_hbm, invcnt_hbm, mean_hbm,
          idx_v, rows_v, acc_v, inv_v, tab_sh, sem0, sem1):
        c = lax.axis_index("c")
        s = lax.axis_index("s")
        sems = (sem0, sem1)
        rslice = pl.ds(s * RT, RT)

        def zero_body(r, carry):
            for q in range(H // 16):
                acc_v[r, pl.ds(q * 16, 16)] = jnp.zeros((16,), jnp.float32)
            return carry

        lax.fori_loop(0, RT, zero_body, 0)
        pltpu.sync_copy(acc_v, tab_sh.at[rslice])
        plsc.subcore_barrier()
        base = c * NP + s * pts_per_tile

        def _ld(st, b):
            off = pl.multiple_of(base + st * _STAGE, _STAGE)
            row = pl.multiple_of((base + st * _STAGE) // CHUNK, _NSUB)
            pltpu.async_copy(index_hbm.at[pl.ds(row, _NSUB)], idx_v.at[b],
                             sems[b])
            pltpu.async_copy(feat_hbm.at[pl.ds(off, _STAGE), pl.ds(0, H)],
                             rows_v.at[b], sems[b])

        def _ld_wait(st, b):
            off = pl.multiple_of(base + st * _STAGE, _STAGE)
            row = pl.multiple_of((base + st * _STAGE) // CHUNK, _NSUB)
            pltpu.make_async_copy(index_hbm.at[pl.ds(row, _NSUB)],
                                  idx_v.at[b], sems[b]).wait()
            pltpu.make_async_copy(feat_hbm.at[pl.ds(off, _STAGE),
                                              pl.ds(0, H)],
                                  rows_v.at[b], sems[b]).wait()

        _ld(0, 0)
        _ld(1, 1)

        def sc_body(g, carry):
            for b in range(2):
                st = g * 2 + b
                _ld_wait(st, b)
                for j in range(_NSUB):
                    pltpu.sync_copy(
                        rows_v.at[b].at[pl.ds(j * CHUNK, CHUNK)],
                        tab_sh.at[idx_v.at[b].at[j]], add=True)
                nxt = st + 2

                @pl.when(nxt < nst)
                def _():
                    _ld(nxt, b)
            return carry

        lax.fori_loop(0, nst // 2, sc_body, 0)
        plsc.subcore_barrier()
        pltpu.sync_copy(tab_sh.at[rslice], acc_v)
        pltpu.sync_copy(invcnt_hbm.at[c].at[s], inv_v)

        def grp_body(g, carry):
            inv16 = inv_v[0, pl.ds(g * 16, 16)]
            for j in range(16):
                bc = jnp.full((16,), inv16[j], jnp.float32)
                r = g * 16 + j
                for q in range(H // 16):
                    cs = pl.ds(q * 16, 16)
                    acc_v[r, cs] = acc_v[r, cs] * bc
            return carry

        lax.fori_loop(0, RT // 16, grp_body, 0)

        @pl.when(s < NTILES - 1)
        def _():
            pltpu.sync_copy(acc_v, mean_hbm.at[pl.ds(c * NX + s * RT, RT)])

        @pl.when(s == NTILES - 1)
        def _():
            pltpu.sync_copy(acc_v.at[pl.ds(0, tail)],
                            mean_hbm.at[pl.ds(c * NX + s * RT, tail)])

    return k(feat, index2d, invcnt)


_TC_BLK = 8192


def _full_spec(shape):
    nd = len(shape)
    return pl.BlockSpec(shape, lambda i: (0,) * nd)


def _full_block_spec():
    return pl.BlockSpec((_TC_BLK, HP), lambda i: (i, 0))


def _tc_first(coordt, wp, bp, w0, b0, w1, b1, ws):
    N = coordt.shape[1]

    def body(cf_ref, wp_ref, bp_ref, w0_ref, b0_ref, w1_ref, b1_ref, ws_ref,
             out_ref):
        cf = cf_ref[...]
        pp = 2.0 * (cf - jnp.floor(cf) - 0.5)
        x = lax.dot_general(pp, wp_ref[...],
                            (((0,), (0,)), ((), ())),
                            preferred_element_type=jnp.float32) + bp_ref[...]
        h = jnp.dot(_gelu(x), w0_ref[...],
                    preferred_element_type=jnp.float32) + b0_ref[...]
        dx = jnp.dot(_gelu(h), w1_ref[...],
                     preferred_element_type=jnp.float32) + b1_ref[...]
        o = jnp.dot(x, ws_ref[...],
                    preferred_element_type=jnp.float32) + dx
        out_ref[...] = jnp.concatenate(
            [o, jnp.zeros((o.shape[0], HP - HID), jnp.float32)], axis=1)

    return pl.pallas_call(
        body,
        grid=(N // _TC_BLK,),
        in_specs=[
            pl.BlockSpec((3, _TC_BLK), lambda i: (0, i)),
            _full_spec(wp.shape), _full_spec(bp.shape),
            _full_spec(w0.shape), _full_spec(b0.shape),
            _full_spec(w1.shape), _full_spec(b1.shape),
            _full_spec(ws.shape),
        ],
        out_specs=_full_block_spec(),
        out_shape=jax.ShapeDtypeStruct((N, HP), jnp.float32),
    )(coordt, wp, bp, w0, b0, w1, b1, ws)


def _tc_block(z, w0, b0, w1, b1, ws, wc=None, bc=None):
    N = z.shape[0]
    final = wc is not None

    def body(*refs):
        z_ref, w0_ref, b0_ref, w1_ref, b1_ref, ws_ref = refs[:6]
        out_ref = refs[-1]
        x = z_ref[...]
        h = jnp.dot(_gelu(x), w0_ref[...],
                    preferred_element_type=jnp.float32) + b0_ref[...]
        dx = jnp.dot(_gelu(h), w1_ref[...],
                     preferred_element_type=jnp.float32) + b1_ref[...]
        o = jnp.dot(x, ws_ref[...],
                    preferred_element_type=jnp.float32) + dx
        if final:
            wc_ref, bc_ref = refs[6], refs[7]
            o = jnp.dot(o, wc_ref[...],
                        preferred_element_type=jnp.float32) + bc_ref[...]
        out_ref[...] = jnp.concatenate(
            [o, jnp.zeros((o.shape[0], HP - HID), jnp.float32)], axis=1)

    args = [z, w0, b0, w1, b1, ws]
    if final:
        args += [wc, bc]
    in_specs = [_full_block_spec()] + [_full_spec(a.shape) for a in args[1:]]
    return pl.pallas_call(
        body,
        grid=(N // _TC_BLK,),
        in_specs=in_specs,
        out_specs=_full_block_spec(),
        out_shape=jax.ShapeDtypeStruct((N, HP), jnp.float32),
    )(*args)


def kernel(p, sparse_coords, res, params):
    Bn, NP, _ = p.shape
    N = Bn * NP
    NX = sparse_coords.shape[0] // Bn

    dat = jnp.clip(p + 0.5, 1e-6, 1.0 - 1e-6)
    coord = dat * res
    ci = coord.astype(jnp.int32)
    vox = (ci[..., 0] * res + ci[..., 1]) * res + ci[..., 2]
    lin = (sparse_coords[:, 1] * res + sparse_coords[:, 2]) * res \
        + sparse_coords[:, 3]
    coords = lin.reshape(Bn, NX).astype(jnp.int32)
    coordt = coord.reshape(N, 3).T

    index, invcnt = _index_kernel(vox, coords)
    index2d = index.reshape(N // CHUNK, CHUNK)

    Wp, bp = params["fc_pos"]
    bpr = bp.reshape(1, 2 * HID)

    W0, b0, W1, b1, Ws = params["blocks"][0]
    net = _tc_first(coordt, Wp.T, bpr, W0.T, b0.reshape(1, HID),
                    W1.T, b1.reshape(1, HID), Ws.T)

    Wc, bc = params["fc_c"]
    nblocks = len(params["blocks"])
    for i in range(1, nblocks):
        W0, b0, W1, b1, Ws = params["blocks"][i]
        z = _pool_kernel(net, index2d, invcnt)
        last = i == nblocks - 1
        net = _tc_block(z, W0.T, b0.reshape(1, HID),
                        W1.T, b1.reshape(1, HID), Ws.T,
                        wc=Wc.T if last else None,
                        bc=bc.reshape(1, HID) if last else None)

    return _scatter_mean_kernel(net, index2d, invcnt, NX)


# speedup vs baseline: 1.0947x; 1.0444x over previous
import functools

import jax
import jax.numpy as jnp
from jax import lax
from jax.experimental import pallas as pl
from jax.experimental.pallas import tpu as pltpu
from jax.experimental.pallas import tpu_sc as plsc

HID = 64
HP = 128
NTILES = 16
CHUNK = 128
RT = 528
SIZE_P = RT * NTILES


def _gelu(x):
    return jax.nn.gelu(x, approximate=True)


def _sc_mesh():
    return plsc.VectorSubcoreMesh(core_axis_name="c", subcore_axis_name="s")


_SC_PARAMS = pltpu.CompilerParams(needs_layout_passes=False,
                                  use_tc_tiling_on_sc=False)


def _index_kernel(vox, coords):
    Bn, NP = vox.shape
    NX = coords.shape[1]
    pts_per_tile = NP // NTILES
    nch = pts_per_tile // CHUNK
    steps = []
    st = NX
    while st >= 1:
        steps.append(st)
        st //= 2

    @functools.partial(
        pl.kernel,
        out_type=[
            jax.ShapeDtypeStruct((Bn, NP), jnp.int32),
            jax.ShapeDtypeStruct((Bn, NTILES, 1, RT), jnp.float32),
        ],
        mesh=_sc_mesh(),
        compiler_params=_SC_PARAMS,
        scratch_types=[
            pltpu.VMEM((NX,), jnp.int32),
            pltpu.VMEM((CHUNK,), jnp.int32),
            pltpu.VMEM((CHUNK,), jnp.int32),
            pltpu.VMEM((CHUNK, 16), jnp.float32),
            pltpu.VMEM((RT, 16), jnp.float32),
            pltpu.VMEM((1, RT), jnp.float32),
            pltpu.VMEM_SHARED((SIZE_P, 16), jnp.float32),
        ],
    )
    def k(vox_hbm, coords_hbm, index_hbm, invcnt_hbm,
          coords_v, vox_v, idx_v, ones_v, cnt_v, inv_v, cnt_sh):
        c = lax.axis_index("c")
        s = lax.axis_index("s")
        rslice = pl.ds(s * RT, RT)
        pltpu.sync_copy(coords_hbm.at[c], coords_v)

        def zero_body(r, carry):
            ones_v[r, :] = jnp.ones((16,), jnp.float32)
            cnt_v[r, :] = jnp.zeros((16,), jnp.float32)
            return carry

        lax.fori_loop(0, CHUNK, zero_body, 0)

        def zero_body2(r, carry):
            cnt_v[r, :] = jnp.zeros((16,), jnp.float32)
            return carry

        lax.fori_loop(CHUNK, RT, zero_body2, 0)
        pltpu.sync_copy(cnt_v, cnt_sh.at[rslice])
        plsc.subcore_barrier()
        base = s * pts_per_tile

        def chunk_body(ch, carry):
            off = pl.multiple_of(base + ch * CHUNK, CHUNK)
            pltpu.sync_copy(vox_hbm.at[c].at[pl.ds(off, CHUNK)], vox_v)
            for r in range(CHUNK // 16):
                v = vox_v[pl.ds(r * 16, 16)]
                pos = jnp.zeros((16,), jnp.int32)
                for st in steps:
                    nxt = pos + st
                    ok = nxt <= NX
                    probe = jnp.minimum(nxt - 1, NX - 1)
                    cv = plsc.load_gather(coords_v, [probe])
                    pos = jnp.where(ok & (cv < v), nxt, pos)
                idx_v[pl.ds(r * 16, 16)] = pos
            pltpu.sync_copy(idx_v, index_hbm.at[c].at[pl.ds(off, CHUNK)])
            pltpu.sync_copy(ones_v, cnt_sh.at[idx_v], add=True)
            return carry

        lax.fori_loop(0, nch, chunk_body, 0)
        plsc.subcore_barrier()
        pltpu.sync_copy(cnt_sh.at[rslice], cnt_v)

        def inv_body(g, carry):
            rows = g * 16 + lax.iota(jnp.int32, 16)
            cnt = plsc.load_gather(cnt_v, [rows, jnp.zeros((16,), jnp.int32)])
            inv_v[0, pl.ds(g * 16, 16)] = 1.0 / jnp.maximum(cnt, 1.0)
            return carry

        lax.fori_loop(0, RT // 16, inv_body, 0)
        pltpu.sync_copy(inv_v, invcnt_hbm.at[c].at[s])

    return k(vox, coords)


_STAGE = 256
_NSUB = _STAGE // CHUNK


def _pool_kernel(feat, index2d, invcnt):
    N = feat.shape[0]
    Bn = invcnt.shape[0]
    NP = N // Bn
    pts_per_tile = NP // NTILES
    nst = pts_per_tile // _STAGE
    H = HID

    @functools.partial(
        pl.kernel,
        out_type=jax.ShapeDtypeStruct((N, HP), jnp.float32),
        mesh=_sc_mesh(),
        compiler_params=_SC_PARAMS,
        scratch_types=[
            pltpu.VMEM((pts_per_tile // CHUNK, CHUNK), jnp.int32),
            pltpu.VMEM((2, _STAGE, H), jnp.float32),
            pltpu.VMEM((RT, H), jnp.float32),
            pltpu.VMEM((1, RT), jnp.float32),
            pltpu.VMEM_SHARED((SIZE_P, H), jnp.float32),
            pltpu.SemaphoreType.DMA,
            pltpu.SemaphoreType.DMA,
        ],
    )
    def k(feat_hbm, index_hbm, invcnt_hbm, z_hbm,
          idx_all, rows_v, acc_v, inv_v, tab_sh, sem0, sem1):
        c = lax.axis_index("c")
        s = lax.axis_index("s")
        sems = (sem0, sem1)
        rslice = pl.ds(s * RT, RT)
        nrows = pts_per_tile // CHUNK
        base = c * NP + s * pts_per_tile
        pltpu.async_copy(
            index_hbm.at[pl.ds(pl.multiple_of(base // CHUNK, nrows), nrows)],
            idx_all, sems[0])

        def zero_body(r, carry):
            for q in range(H // 16):
                acc_v[r, pl.ds(q * 16, 16)] = jnp.zeros((16,), jnp.float32)
            return carry

        lax.fori_loop(0, RT, zero_body, 0)
        pltpu.sync_copy(acc_v, tab_sh.at[rslice])
        pltpu.make_async_copy(
            index_hbm.at[pl.ds(pl.multiple_of(base // CHUNK, nrows), nrows)],
            idx_all, sems[0]).wait()
        plsc.subcore_barrier()

        def _ld(st, b):
            off = pl.multiple_of(base + st * _STAGE, _STAGE)
            pltpu.async_copy(feat_hbm.at[pl.ds(off, _STAGE), pl.ds(0, H)],
                             rows_v.at[b], sems[b])

        def _ld_wait(st, b):
            off = pl.multiple_of(base + st * _STAGE, _STAGE)
            pltpu.make_async_copy(feat_hbm.at[pl.ds(off, _STAGE),
                                              pl.ds(0, H)],
                                  rows_v.at[b], sems[b]).wait()

        _ld(0, 0)
        _ld(1, 1)

        def sc_body(g, carry):
            for b in range(2):
                st = g * 2 + b
                off = pl.multiple_of(base + st * _STAGE, _STAGE)
                _ld_wait(st, b)
                for j in range(_NSUB):
                    pltpu.sync_copy(
                        rows_v.at[b].at[pl.ds(j * CHUNK, CHUNK)],
                        tab_sh.at[idx_all.at[st * _NSUB + j]], add=True)
                pltpu.sync_copy(rows_v.at[b],
                                z_hbm.at[pl.ds(off, _STAGE), pl.ds(0, H)])
                nxt = st + 2

                @pl.when(nxt < nst)
                def _():
                    _ld(nxt, b)
            return carry

        lax.fori_loop(0, nst // 2, sc_body, 0)
        plsc.subcore_barrier()
        pltpu.sync_copy(tab_sh.at[rslice], acc_v)
        pltpu.sync_copy(invcnt_hbm.at[c].at[s], inv_v)

        def grp_body(g, carry):
            inv16 = inv_v[0, pl.ds(g * 16, 16)]
            for j in range(16):
                bc = jnp.full((16,), inv16[j], jnp.float32)
                r = g * 16 + j
                for q in range(H // 16):
                    cs = pl.ds(q * 16, 16)
                    acc_v[r, cs] = acc_v[r, cs] * bc
            return carry

        lax.fori_loop(0, RT // 16, grp_body, 0)
        pltpu.sync_copy(acc_v, tab_sh.at[rslice])
        plsc.subcore_barrier()

        def _gst(st, b):
            for j in range(_NSUB):
                pltpu.async_copy(tab_sh.at[idx_all.at[st * _NSUB + j]],
                                 rows_v.at[b].at[pl.ds(j * CHUNK, CHUNK)],
                                 sems[b])

        def _gproc(st, b):
            for j in range(_NSUB):
                pltpu.make_async_copy(
                    tab_sh.at[idx_all.at[st * _NSUB + j]],
                    rows_v.at[b].at[pl.ds(j * CHUNK, CHUNK)],
                    sems[b]).wait()
            off = pl.multiple_of(base + st * _STAGE, _STAGE)
            pltpu.sync_copy(rows_v.at[b],
                            z_hbm.at[pl.ds(off, _STAGE), pl.ds(H, H)])

        _gst(0, 0)
        _gst(1, 1)

        def g_body(g, carry):
            for b in range(2):
                st = g * 2 + b
                _gproc(st, b)
                nxt = st + 2

                @pl.when(nxt < nst)
                def _():
                    _gst(nxt, b)
            return carry

        lax.fori_loop(0, nst // 2, g_body, 0)

    return k(feat, index2d, invcnt)


def _scatter_mean_kernel(feat, index2d, invcnt, NX):
    N = feat.shape[0]
    Bn = invcnt.shape[0]
    NP = N // Bn
    pts_per_tile = NP // NTILES
    nst = pts_per_tile // _STAGE
    H = HID
    tail = NX - (NTILES - 1) * RT
    assert 0 < tail <= RT

    @functools.partial(
        pl.kernel,
        out_type=jax.ShapeDtypeStruct((Bn * NX, H), jnp.float32),
        mesh=_sc_mesh(),
        compiler_params=_SC_PARAMS,
        scratch_types=[
            pltpu.VMEM((2, _NSUB, CHUNK), jnp.int32),
            pltpu.VMEM((2, _STAGE, H), jnp.float32),
            pltpu.VMEM((RT, H), jnp.float32),
            pltpu.VMEM((1, RT), jnp.float32),
            pltpu.VMEM_SHARED((SIZE_P, H), jnp.float32),
            pltpu.SemaphoreType.DMA,
            pltpu.SemaphoreType.DMA,
        ],
    )
    def k(feat_hbm, index_hbm, invcnt_hbm, mean_hbm,
          idx_v, rows_v, acc_v, inv_v, tab_sh, sem0, sem1):
        c = lax.axis_index("c")
        s = lax.axis_index("s")
        sems = (sem0, sem1)
        rslice = pl.ds(s * RT, RT)

        def zero_body(r, carry):
            for q in range(H // 16):
                acc_v[r, pl.ds(q * 16, 16)] = jnp.zeros((16,), jnp.float32)
            return carry

        lax.fori_loop(0, RT, zero_body, 0)
        pltpu.sync_copy(acc_v, tab_sh.at[rslice])
        plsc.subcore_barrier()
        base = c * NP + s * pts_per_tile

        def _ld(st, b):
            off = pl.multiple_of(base + st * _STAGE, _STAGE)
            row = pl.multiple_of((base + st * _STAGE) // CHUNK, _NSUB)
            pltpu.async_copy(index_hbm.at[pl.ds(row, _NSUB)], idx_v.at[b],
                             sems[b])
            pltpu.async_copy(feat_hbm.at[pl.ds(off, _STAGE), pl.ds(0, H)],
                             rows_v.at[b], sems[b])

        def _ld_wait(st, b):
            off = pl.multiple_of(base + st * _STAGE, _STAGE)
            row = pl.multiple_of((base + st * _STAGE) // CHUNK, _NSUB)
            pltpu.make_async_copy(index_hbm.at[pl.ds(row, _NSUB)],
                                  idx_v.at[b], sems[b]).wait()
            pltpu.make_async_copy(feat_hbm.at[pl.ds(off, _STAGE),
                                              pl.ds(0, H)],
                                  rows_v.at[b], sems[b]).wait()

        _ld(0, 0)
        _ld(1, 1)

        def sc_body(g, carry):
            for b in range(2):
                st = g * 2 + b
                _ld_wait(st, b)
                for j in range(_NSUB):
                    pltpu.sync_copy(
                        rows_v.at[b].at[pl.ds(j * CHUNK, CHUNK)],
                        tab_sh.at[idx_v.at[b].at[j]], add=True)
                nxt = st + 2

                @pl.when(nxt < nst)
                def _():
                    _ld(nxt, b)
            return carry

        lax.fori_loop(0, nst // 2, sc_body, 0)
        plsc.subcore_barrier()
        pltpu.sync_copy(tab_sh.at[rslice], acc_v)
        pltpu.sync_copy(invcnt_hbm.at[c].at[s], inv_v)

        def grp_body(g, carry):
            inv16 = inv_v[0, pl.ds(g * 16, 16)]
            for j in range(16):
                bc = jnp.full((16,), inv16[j], jnp.float32)
                r = g * 16 + j
                for q in range(H // 16):
                    cs = pl.ds(q * 16, 16)
                    acc_v[r, cs] = acc_v[r, cs] * bc
            return carry

        lax.fori_loop(0, RT // 16, grp_body, 0)

        @pl.when(s < NTILES - 1)
        def _():
            pltpu.sync_copy(acc_v, mean_hbm.at[pl.ds(c * NX + s * RT, RT)])

        @pl.when(s == NTILES - 1)
        def _():
            pltpu.sync_copy(acc_v.at[pl.ds(0, tail)],
                            mean_hbm.at[pl.ds(c * NX + s * RT, tail)])

    return k(feat, index2d, invcnt)


_TC_BLK = 8192


def _full_spec(shape):
    nd = len(shape)
    return pl.BlockSpec(shape, lambda i: (0,) * nd)


def _full_block_spec():
    return pl.BlockSpec((_TC_BLK, HP), lambda i: (i, 0))


def _tc_first(coordt, wp, bp, w0, b0, w1, b1, ws):
    N = coordt.shape[1]

    def body(cf_ref, wp_ref, bp_ref, w0_ref, b0_ref, w1_ref, b1_ref, ws_ref,
             out_ref):
        cf = cf_ref[...]
        pp = 2.0 * (cf - jnp.floor(cf) - 0.5)
        x = lax.dot_general(pp, wp_ref[...],
                            (((0,), (0,)), ((), ())),
                            preferred_element_type=jnp.float32) + bp_ref[...]
        h = jnp.dot(_gelu(x), w0_ref[...],
                    preferred_element_type=jnp.float32) + b0_ref[...]
        dx = jnp.dot(_gelu(h), w1_ref[...],
                     preferred_element_type=jnp.float32) + b1_ref[...]
        o = jnp.dot(x, ws_ref[...],
                    preferred_element_type=jnp.float32) + dx
        out_ref[...] = jnp.concatenate(
            [o, jnp.zeros((o.shape[0], HP - HID), jnp.float32)], axis=1)

    return pl.pallas_call(
        body,
        grid=(N // _TC_BLK,),
        in_specs=[
            pl.BlockSpec((3, _TC_BLK), lambda i: (0, i)),
            _full_spec(wp.shape), _full_spec(bp.shape),
            _full_spec(w0.shape), _full_spec(b0.shape),
            _full_spec(w1.shape), _full_spec(b1.shape),
            _full_spec(ws.shape),
        ],
        out_specs=_full_block_spec(),
        out_shape=jax.ShapeDtypeStruct((N, HP), jnp.float32),
    )(coordt, wp, bp, w0, b0, w1, b1, ws)


def _tc_block(z, w0, b0, w1, b1, ws, wc=None, bc=None):
    N = z.shape[0]
    final = wc is not None

    def body(*refs):
        z_ref, w0_ref, b0_ref, w1_ref, b1_ref, ws_ref = refs[:6]
        out_ref = refs[-1]
        x = z_ref[...]
        h = jnp.dot(_gelu(x), w0_ref[...],
                    preferred_element_type=jnp.float32) + b0_ref[...]
        dx = jnp.dot(_gelu(h), w1_ref[...],
                     preferred_element_type=jnp.float32) + b1_ref[...]
        o = jnp.dot(x, ws_ref[...],
                    preferred_element_type=jnp.float32) + dx
        if final:
            wc_ref, bc_ref = refs[6], refs[7]
            o = jnp.dot(o, wc_ref[...],
                        preferred_element_type=jnp.float32) + bc_ref[...]
        out_ref[...] = jnp.concatenate(
            [o, jnp.zeros((o.shape[0], HP - HID), jnp.float32)], axis=1)

    args = [z, w0, b0, w1, b1, ws]
    if final:
        args += [wc, bc]
    in_specs = [_full_block_spec()] + [_full_spec(a.shape) for a in args[1:]]
    return pl.pallas_call(
        body,
        grid=(N // _TC_BLK,),
        in_specs=in_specs,
        out_specs=_full_block_spec(),
        out_shape=jax.ShapeDtypeStruct((N, HP), jnp.float32),
    )(*args)


def kernel(p, sparse_coords, res, params):
    Bn, NP, _ = p.shape
    N = Bn * NP
    NX = sparse_coords.shape[0] // Bn

    dat = jnp.clip(p + 0.5, 1e-6, 1.0 - 1e-6)
    coord = dat * res
    ci = coord.astype(jnp.int32)
    vox = (ci[..., 0] * res + ci[..., 1]) * res + ci[..., 2]
    lin = (sparse_coords[:, 1] * res + sparse_coords[:, 2]) * res \
        + sparse_coords[:, 3]
    coords = lin.reshape(Bn, NX).astype(jnp.int32)
    coordt = coord.reshape(N, 3).T

    index, invcnt = _index_kernel(vox, coords)
    index2d = index.reshape(N // CHUNK, CHUNK)

    Wp, bp = params["fc_pos"]
    bpr = bp.reshape(1, 2 * HID)

    W0, b0, W1, b1, Ws = params["blocks"][0]
    net = _tc_first(coordt, Wp.T, bpr, W0.T, b0.reshape(1, HID),
                    W1.T, b1.reshape(1, HID), Ws.T)

    Wc, bc = params["fc_c"]
    nblocks = len(params["blocks"])
    for i in range(1, nblocks):
        W0, b0, W1, b1, Ws = params["blocks"][i]
        z = _pool_kernel(net, index2d, invcnt)
        last = i == nblocks - 1
        net = _tc_block(z, W0.T, b0.reshape(1, HID),
                        W1.T, b1.reshape(1, HID), Ws.T,
                        wc=Wc.T if last else None,
                        bc=bc.reshape(1, HID) if last else None)

    return _scatter_mean_kernel(net, index2d, invcnt, NX)


# speedup vs baseline: 1.0952x; 1.0005x over previous
import functools

import jax
import jax.numpy as jnp
from jax import lax
from jax.experimental import pallas as pl
from jax.experimental.pallas import tpu as pltpu
from jax.experimental.pallas import tpu_sc as plsc

HID = 64
HP = 128
NTILES = 16
CHUNK = 128
RT = 528
SIZE_P = RT * NTILES


def _gelu(x):
    return jax.nn.gelu(x, approximate=True)


def _sc_mesh():
    return plsc.VectorSubcoreMesh(core_axis_name="c", subcore_axis_name="s")


_SC_PARAMS = pltpu.CompilerParams(needs_layout_passes=False,
                                  use_tc_tiling_on_sc=False)


def _index_kernel(vox, coords):
    Bn, NP = vox.shape
    NX = coords.shape[1]
    pts_per_tile = NP // NTILES
    nch = pts_per_tile // CHUNK
    steps = []
    st = NX
    while st >= 1:
        steps.append(st)
        st //= 2

    @functools.partial(
        pl.kernel,
        out_type=[
            jax.ShapeDtypeStruct((Bn, NP), jnp.int32),
            jax.ShapeDtypeStruct((Bn, NTILES, 1, RT), jnp.float32),
        ],
        mesh=_sc_mesh(),
        compiler_params=_SC_PARAMS,
        scratch_types=[
            pltpu.VMEM((NX,), jnp.int32),
            pltpu.VMEM((CHUNK,), jnp.int32),
            pltpu.VMEM((CHUNK,), jnp.int32),
            pltpu.VMEM((CHUNK, 16), jnp.float32),
            pltpu.VMEM((RT, 16), jnp.float32),
            pltpu.VMEM((1, RT), jnp.float32),
            pltpu.VMEM_SHARED((SIZE_P, 16), jnp.float32),
        ],
    )
    def k(vox_hbm, coords_hbm, index_hbm, invcnt_hbm,
          coords_v, vox_v, idx_v, ones_v, cnt_v, inv_v, cnt_sh):
        c = lax.axis_index("c")
        s = lax.axis_index("s")
        rslice = pl.ds(s * RT, RT)
        pltpu.sync_copy(coords_hbm.at[c], coords_v)

        def zero_body(r, carry):
            ones_v[r, :] = jnp.ones((16,), jnp.float32)
            cnt_v[r, :] = jnp.zeros((16,), jnp.float32)
            return carry

        lax.fori_loop(0, CHUNK, zero_body, 0)

        def zero_body2(r, carry):
            cnt_v[r, :] = jnp.zeros((16,), jnp.float32)
            return carry

        lax.fori_loop(CHUNK, RT, zero_body2, 0)
        pltpu.sync_copy(cnt_v, cnt_sh.at[rslice])
        plsc.subcore_barrier()
        base = s * pts_per_tile

        def chunk_body(ch, carry):
            off = pl.multiple_of(base + ch * CHUNK, CHUNK)
            pltpu.sync_copy(vox_hbm.at[c].at[pl.ds(off, CHUNK)], vox_v)
            for r in range(CHUNK // 16):
                v = vox_v[pl.ds(r * 16, 16)]
                pos = jnp.zeros((16,), jnp.int32)
                for st in steps:
                    nxt = pos + st
                    ok = nxt <= NX
                    probe = jnp.minimum(nxt - 1, NX - 1)
                    cv = plsc.load_gather(coords_v, [probe])
                    pos = jnp.where(ok & (cv < v), nxt, pos)
                idx_v[pl.ds(r * 16, 16)] = pos
            pltpu.sync_copy(idx_v, index_hbm.at[c].at[pl.ds(off, CHUNK)])
            pltpu.sync_copy(ones_v, cnt_sh.at[idx_v], add=True)
            return carry

        lax.fori_loop(0, nch, chunk_body, 0)
        plsc.subcore_barrier()
        pltpu.sync_copy(cnt_sh.at[rslice], cnt_v)

        def inv_body(g, carry):
            rows = g * 16 + lax.iota(jnp.int32, 16)
            cnt = plsc.load_gather(cnt_v, [rows, jnp.zeros((16,), jnp.int32)])
            inv_v[0, pl.ds(g * 16, 16)] = 1.0 / jnp.maximum(cnt, 1.0)
            return carry

        lax.fori_loop(0, RT // 16, inv_body, 0)
        pltpu.sync_copy(inv_v, invcnt_hbm.at[c].at[s])

    return k(vox, coords)


_STAGE = 256
_NSUB = _STAGE // CHUNK


def _pool_kernel(feat, index2d, invcnt):
    N = feat.shape[0]
    Bn = invcnt.shape[0]
    NP = N // Bn
    pts_per_tile = NP // NTILES
    nst = pts_per_tile // _STAGE
    H = HID

    @functools.partial(
        pl.kernel,
        out_type=jax.ShapeDtypeStruct((N, HP), jnp.float32),
        mesh=_sc_mesh(),
        compiler_params=_SC_PARAMS,
        scratch_types=[
            pltpu.VMEM((pts_per_tile // CHUNK, CHUNK), jnp.int32),
            pltpu.VMEM((2, _STAGE, H), jnp.float32),
            pltpu.VMEM((RT, H), jnp.float32),
            pltpu.VMEM((1, RT), jnp.float32),
            pltpu.VMEM_SHARED((SIZE_P, H), jnp.float32),
            pltpu.SemaphoreType.DMA,
            pltpu.SemaphoreType.DMA,
        ],
    )
    def k(feat_hbm, index_hbm, invcnt_hbm, z_hbm,
          idx_all, rows_v, acc_v, inv_v, tab_sh, sem0, sem1):
        c = lax.axis_index("c")
        s = lax.axis_index("s")
        sems = (sem0, sem1)
        rslice = pl.ds(s * RT, RT)
        nrows = pts_per_tile // CHUNK
        base = c * NP + s * pts_per_tile
        pltpu.async_copy(
            index_hbm.at[pl.ds(pl.multiple_of(base // CHUNK, nrows), nrows)],
            idx_all, sems[0])

        def zero_body(r, carry):
            for q in range(H // 16):
                acc_v[r, pl.ds(q * 16, 16)] = jnp.zeros((16,), jnp.float32)
            return carry

        lax.fori_loop(0, RT, zero_body, 0)
        pltpu.sync_copy(acc_v, tab_sh.at[rslice])
        pltpu.make_async_copy(
            index_hbm.at[pl.ds(pl.multiple_of(base // CHUNK, nrows), nrows)],
            idx_all, sems[0]).wait()
        plsc.subcore_barrier()

        def _ld(st, b):
            off = pl.multiple_of(base + st * _STAGE, _STAGE)
            pltpu.async_copy(feat_hbm.at[pl.ds(off, _STAGE), pl.ds(0, H)],
                             rows_v.at[b], sems[b])

        def _ld_wait(st, b):
            off = pl.multiple_of(base + st * _STAGE, _STAGE)
            pltpu.make_async_copy(feat_hbm.at[pl.ds(off, _STAGE),
                                              pl.ds(0, H)],
                                  rows_v.at[b], sems[b]).wait()

        _ld(0, 0)
        _ld(1, 1)

        def sc_body(g, carry):
            for b in range(2):
                st = g * 2 + b
                off = pl.multiple_of(base + st * _STAGE, _STAGE)
                _ld_wait(st, b)
                for j in range(_NSUB):
                    pltpu.sync_copy(
                        rows_v.at[b].at[pl.ds(j * CHUNK, CHUNK)],
                        tab_sh.at[idx_all.at[st * _NSUB + j]], add=True)
                pltpu.sync_copy(rows_v.at[b],
                                z_hbm.at[pl.ds(off, _STAGE), pl.ds(0, H)])
                nxt = st + 2

                @pl.when(nxt < nst)
                def _():
                    _ld(nxt, b)
            return carry

        lax.fori_loop(0, nst // 2, sc_body, 0)
        plsc.subcore_barrier()
        pltpu.sync_copy(tab_sh.at[rslice], acc_v)
        pltpu.sync_copy(invcnt_hbm.at[c].at[s], inv_v)

        def grp_body(g, carry):
            inv16 = inv_v[0, pl.ds(g * 16, 16)]
            for j in range(16):
                bc = jnp.full((16,), inv16[j], jnp.float32)
                r = g * 16 + j
                for q in range(H // 16):
                    cs = pl.ds(q * 16, 16)
                    acc_v[r, cs] = acc_v[r, cs] * bc
            return carry

        lax.fori_loop(0, RT // 16, grp_body, 0)
        pltpu.sync_copy(acc_v, tab_sh.at[rslice])
        plsc.subcore_barrier()

        def _gst(st, b):
            for j in range(_NSUB):
                pltpu.async_copy(tab_sh.at[idx_all.at[st * _NSUB + j]],
                                 rows_v.at[b].at[pl.ds(j * CHUNK, CHUNK)],
                                 sems[b])

        def _gproc(st, b):
            for j in range(_NSUB):
                pltpu.make_async_copy(
                    tab_sh.at[idx_all.at[st * _NSUB + j]],
                    rows_v.at[b].at[pl.ds(j * CHUNK, CHUNK)],
                    sems[b]).wait()
            off = pl.multiple_of(base + st * _STAGE, _STAGE)
            pltpu.sync_copy(rows_v.at[b],
                            z_hbm.at[pl.ds(off, _STAGE), pl.ds(H, H)])

        _gst(0, 0)
        _gst(1, 1)

        def g_body(g, carry):
            for b in range(2):
                st = g * 2 + b
                _gproc(st, b)
                nxt = st + 2

                @pl.when(nxt < nst)
                def _():
                    _gst(nxt, b)
            return carry

        lax.fori_loop(0, nst // 2, g_body, 0)

    return k(feat, index2d, invcnt)


def _scatter_mean_kernel(feat, index2d, invcnt, NX):
    N = feat.shape[0]
    Bn = invcnt.shape[0]
    NP = N // Bn
    pts_per_tile = NP // NTILES
    nst = pts_per_tile // _STAGE
    H = HID
    tail = NX - (NTILES - 1) * RT
    assert 0 < tail <= RT

    @functools.partial(
        pl.kernel,
        out_type=jax.ShapeDtypeStruct((Bn * NX, H), jnp.float32),
        mesh=_sc_mesh(),
        compiler_params=_SC_PARAMS,
        scratch_types=[
            pltpu.VMEM((pts_per_tile // CHUNK, CHUNK), jnp.int32),
            pltpu.VMEM((2, _STAGE, H), jnp.float32),
            pltpu.VMEM((RT, H), jnp.float32),
            pltpu.VMEM((1, RT), jnp.float32),
            pltpu.VMEM_SHARED((SIZE_P, H), jnp.float32),
            pltpu.SemaphoreType.DMA,
            pltpu.SemaphoreType.DMA,
        ],
    )
    def k(feat_hbm, index_hbm, invcnt_hbm, mean_hbm,
          idx_all, rows_v, acc_v, inv_v, tab_sh, sem0, sem1):
        c = lax.axis_index("c")
        s = lax.axis_index("s")
        sems = (sem0, sem1)
        rslice = pl.ds(s * RT, RT)
        nrows = pts_per_tile // CHUNK
        base = c * NP + s * pts_per_tile
        pltpu.async_copy(
            index_hbm.at[pl.ds(pl.multiple_of(base // CHUNK, nrows), nrows)],
            idx_all, sems[0])

        def zero_body(r, carry):
            for q in range(H // 16):
                acc_v[r, pl.ds(q * 16, 16)] = jnp.zeros((16,), jnp.float32)
            return carry

        lax.fori_loop(0, RT, zero_body, 0)
        pltpu.sync_copy(acc_v, tab_sh.at[rslice])
        pltpu.make_async_copy(
            index_hbm.at[pl.ds(pl.multiple_of(base // CHUNK, nrows), nrows)],
            idx_all, sems[0]).wait()
        plsc.subcore_barrier()

        def _ld(st, b):
            off = pl.multiple_of(base + st * _STAGE, _STAGE)
            pltpu.async_copy(feat_hbm.at[pl.ds(off, _STAGE), pl.ds(0, H)],
                             rows_v.at[b], sems[b])

        def _ld_wait(st, b):
            off = pl.multiple_of(base + st * _STAGE, _STAGE)
            pltpu.make_async_copy(feat_hbm.at[pl.ds(off, _STAGE),
                                              pl.ds(0, H)],
                                  rows_v.at[b], sems[b]).wait()

        _ld(0, 0)
        _ld(1, 1)

        def sc_body(g, carry):
            for b in range(2):
                st = g * 2 + b
                _ld_wait(st, b)
                for j in range(_NSUB):
                    pltpu.sync_copy(
                        rows_v.at[b].at[pl.ds(j * CHUNK, CHUNK)],
                        tab_sh.at[idx_all.at[st * _NSUB + j]], add=True)
                nxt = st + 2

                @pl.when(nxt < nst)
                def _():
                    _ld(nxt, b)
            return carry

        lax.fori_loop(0, nst // 2, sc_body, 0)
        plsc.subcore_barrier()
        pltpu.sync_copy(tab_sh.at[rslice], acc_v)
        pltpu.sync_copy(invcnt_hbm.at[c].at[s], inv_v)

        def grp_body(g, carry):
            inv16 = inv_v[0, pl.ds(g * 16, 16)]
            for j in range(16):
                bc = jnp.full((16,), inv16[j], jnp.float32)
                r = g * 16 + j
                for q in range(H // 16):
                    cs = pl.ds(q * 16, 16)
                    acc_v[r, cs] = acc_v[r, cs] * bc
            return carry

        lax.fori_loop(0, RT // 16, grp_body, 0)

        @pl.when(s < NTILES - 1)
        def _():
            pltpu.sync_copy(acc_v, mean_hbm.at[pl.ds(c * NX + s * RT, RT)])

        @pl.when(s == NTILES - 1)
        def _():
            pltpu.sync_copy(acc_v.at[pl.ds(0, tail)],
                            mean_hbm.at[pl.ds(c * NX + s * RT, tail)])

    return k(feat, index2d, invcnt)


_TC_BLK = 8192


def _full_spec(shape):
    nd = len(shape)
    return pl.BlockSpec(shape, lambda i: (0,) * nd)


def _full_block_spec():
    return pl.BlockSpec((_TC_BLK, HP), lambda i: (i, 0))


def _tc_first(coordt, wp, bp, w0, b0, w1, b1, ws):
    N = coordt.shape[1]

    def body(cf_ref, wp_ref, bp_ref, w0_ref, b0_ref, w1_ref, b1_ref, ws_ref,
             out_ref):
        cf = cf_ref[...]
        pp = 2.0 * (cf - jnp.floor(cf) - 0.5)
        x = lax.dot_general(pp, wp_ref[...],
                            (((0,), (0,)), ((), ())),
                            preferred_element_type=jnp.float32) + bp_ref[...]
        h = jnp.dot(_gelu(x), w0_ref[...],
                    preferred_element_type=jnp.float32) + b0_ref[...]
        dx = jnp.dot(_gelu(h), w1_ref[...],
                     preferred_element_type=jnp.float32) + b1_ref[...]
        o = jnp.dot(x, ws_ref[...],
                    preferred_element_type=jnp.float32) + dx
        out_ref[...] = jnp.concatenate(
            [o, jnp.zeros((o.shape[0], HP - HID), jnp.float32)], axis=1)

    return pl.pallas_call(
        body,
        grid=(N // _TC_BLK,),
        in_specs=[
            pl.BlockSpec((3, _TC_BLK), lambda i: (0, i)),
            _full_spec(wp.shape), _full_spec(bp.shape),
            _full_spec(w0.shape), _full_spec(b0.shape),
            _full_spec(w1.shape), _full_spec(b1.shape),
            _full_spec(ws.shape),
        ],
        out_specs=_full_block_spec(),
        out_shape=jax.ShapeDtypeStruct((N, HP), jnp.float32),
    )(coordt, wp, bp, w0, b0, w1, b1, ws)


def _tc_block(z, w0, b0, w1, b1, ws, wc=None, bc=None):
    N = z.shape[0]
    final = wc is not None

    def body(*refs):
        z_ref, w0_ref, b0_ref, w1_ref, b1_ref, ws_ref = refs[:6]
        out_ref = refs[-1]
        x = z_ref[...]
        h = jnp.dot(_gelu(x), w0_ref[...],
                    preferred_element_type=jnp.float32) + b0_ref[...]
        dx = jnp.dot(_gelu(h), w1_ref[...],
                     preferred_element_type=jnp.float32) + b1_ref[...]
        o = jnp.dot(x, ws_ref[...],
                    preferred_element_type=jnp.float32) + dx
        if final:
            wc_ref, bc_ref = refs[6], refs[7]
            o = jnp.dot(o, wc_ref[...],
                        preferred_element_type=jnp.float32) + bc_ref[...]
        out_ref[...] = jnp.concatenate(
            [o, jnp.zeros((o.shape[0], HP - HID), jnp.float32)], axis=1)

    args = [z, w0, b0, w1, b1, ws]
    if final:
        args += [wc, bc]
    in_specs = [_full_block_spec()] + [_full_spec(a.shape) for a in args[1:]]
    return pl.pallas_call(
        body,
        grid=(N // _TC_BLK,),
        in_specs=in_specs,
        out_specs=_full_block_spec(),
        out_shape=jax.ShapeDtypeStruct((N, HP), jnp.float32),
    )(*args)


def kernel(p, sparse_coords, res, params):
    Bn, NP, _ = p.shape
    N = Bn * NP
    NX = sparse_coords.shape[0] // Bn

    dat = jnp.clip(p + 0.5, 1e-6, 1.0 - 1e-6)
    coord = dat * res
    ci = coord.astype(jnp.int32)
    vox = (ci[..., 0] * res + ci[..., 1]) * res + ci[..., 2]
    lin = (sparse_coords[:, 1] * res + sparse_coords[:, 2]) * res \
        + sparse_coords[:, 3]
    coords = lin.reshape(Bn, NX).astype(jnp.int32)
    coordt = coord.reshape(N, 3).T

    index, invcnt = _index_kernel(vox, coords)
    index2d = index.reshape(N // CHUNK, CHUNK)

    Wp, bp = params["fc_pos"]
    bpr = bp.reshape(1, 2 * HID)

    W0, b0, W1, b1, Ws = params["blocks"][0]
    net = _tc_first(coordt, Wp.T, bpr, W0.T, b0.reshape(1, HID),
                    W1.T, b1.reshape(1, HID), Ws.T)

    Wc, bc = params["fc_c"]
    nblocks = len(params["blocks"])
    for i in range(1, nblocks):
        W0, b0, W1, b1, Ws = params["blocks"][i]
        z = _pool_kernel(net, index2d, invcnt)
        last = i == nblocks - 1
        net = _tc_block(z, W0.T, b0.reshape(1, HID),
                        W1.T, b1.reshape(1, HID), Ws.T,
                        wc=Wc.T if last else None,
                        bc=bc.reshape(1, HID) if last else None)

    return _scatter_mean_kernel(net, index2d, invcnt, NX)
